# baseline (speedup 1.0000x reference)
import functools

import jax
import jax.numpy as jnp
from jax.experimental import pallas as pl


def _fc_kernel(seq_ref, wt_ref, out_ref):
    out_ref[...] = jnp.dot(seq_ref[...], wt_ref[...],
                           preferred_element_type=jnp.float32)


def _agg_kernel(adj_ref, fts_ref, b_ref, a_ref, out_ref):
    acc = jnp.dot(adj_ref[...], fts_ref[...],
                  preferred_element_type=jnp.float32)
    acc = acc + b_ref[...]
    out_ref[...] = jnp.where(acc >= 0, acc, a_ref[...] * acc)


@jax.jit
def kernel(seq, adj, W, b, prelu_a):
    _, n, d_in = seq.shape
    d_out = W.shape[0]
    seq2 = seq[0]
    adj2 = adj[0]

    seq_fts = pl.pallas_call(
        _fc_kernel,
        out_shape=jax.ShapeDtypeStruct((n, d_out), jnp.float32),
    )(seq2, W.T)

    blk = 400 if n % 400 == 0 else n
    grid = n // blk

    out = pl.pallas_call(
        _agg_kernel,
        grid=(grid,),
        in_specs=[
            pl.BlockSpec((blk, n), lambda i: (i, 0)),
            pl.BlockSpec((n, d_out), lambda i: (0, 0)),
            pl.BlockSpec((1, d_out), lambda i: (0, 0)),
            pl.BlockSpec((1, d_out), lambda i: (0, 0)),
        ],
        out_specs=pl.BlockSpec((blk, d_out), lambda i: (i, 0)),
        out_shape=jax.ShapeDtypeStruct((n, d_out), jnp.float32),
    )(adj2, seq_fts, b.reshape(1, d_out),
      jnp.full((1, d_out), prelu_a, dtype=jnp.float32))

    return out[None]


# speedup vs baseline: 1.0417x; 1.0417x over previous
import functools

import jax
import jax.numpy as jnp
from jax.experimental import pallas as pl
from jax.experimental.pallas import tpu as pltpu


def _gcn_kernel(seq_ref, wt_ref, adj_ref, b_ref, a_ref, out_ref, fts_ref):
    @pl.when(pl.program_id(0) == 0)
    def _():
        fts_ref[...] = jnp.dot(seq_ref[...], wt_ref[...],
                               preferred_element_type=jnp.float32)

    acc = jnp.dot(adj_ref[...], fts_ref[...],
                  preferred_element_type=jnp.float32)
    acc = acc + b_ref[...]
    out_ref[...] = jnp.where(acc >= 0, acc, a_ref[...] * acc)


@jax.jit
def kernel(seq, adj, W, b, prelu_a):
    _, n, d_in = seq.shape
    d_out = W.shape[0]
    seq2 = seq[0]
    adj2 = adj[0]

    blk = 400 if n % 400 == 0 else n
    grid = n // blk

    out = pl.pallas_call(
        _gcn_kernel,
        grid=(grid,),
        in_specs=[
            pl.BlockSpec((n, d_in), lambda i: (0, 0)),
            pl.BlockSpec((d_in, d_out), lambda i: (0, 0)),
            pl.BlockSpec((blk, n), lambda i: (i, 0)),
            pl.BlockSpec((1, d_out), lambda i: (0, 0)),
            pl.BlockSpec((1, d_out), lambda i: (0, 0)),
        ],
        out_specs=pl.BlockSpec((blk, d_out), lambda i: (i, 0)),
        out_shape=jax.ShapeDtypeStruct((n, d_out), jnp.float32),
        scratch_shapes=[pltpu.VMEM((n, d_out), jnp.float32)],
    )(seq2, W.T, adj2, b.reshape(1, d_out),
      jnp.full((1, d_out), prelu_a, dtype=jnp.float32))

    return out[None]
